# Initial kernel scaffold; baseline (speedup 1.0000x reference)
import jax
import jax.numpy as jnp
from jax.experimental import pallas as pl


def kernel(users, items, user_table, item_table, adj_row, adj_col, adj_val):
    raise NotImplementedError("write your pallas kernel here")


# speedup vs baseline: 6.3543x; 6.3543x over previous
import functools

import jax
import jax.numpy as jnp
from jax import lax
from jax.experimental import pallas as pl
from jax.experimental.pallas import tpu as pltpu
from jax.experimental.pallas import tpu_sc as plsc

NU = 3000
NI = 7000
N = NU + NI
E = 320000
D = 128
NLAYERS = 5
NC = 2
NS = 16
K = 80
NCHUNK = E // (NC * NS * K)
R = 40
CH_U = NU // R
CH_I = NI // R
BIDX = 2048 // (NC * NS)

_mesh = plsc.VectorSubcoreMesh(
    core_axis_name="c", subcore_axis_name="s", num_cores=NC, num_subcores=NS
)


def _fill2d(ref, nrows, value):
    g = ref.shape[1] // 16

    def body(i, _):
        for j in range(g):
            ref[i, pl.ds(j * 16, 16)] = jnp.full((16,), value, jnp.float32)
        return 0

    lax.fori_loop(0, nrows, body, 0)


def _half_loop(c, s, body):
    base = c * NU
    nch = CH_U + c * (CH_I - CH_U)
    n = (nch - 1 - s) // NS + 1

    def wrap(j, _):
        rl = (s + j * NS) * R
        body(base + rl, rl)
        return 0

    lax.fori_loop(0, n, wrap, 0)


@functools.partial(
    pl.kernel,
    out_type=jax.ShapeDtypeStruct((N, D), jnp.float32),
    mesh=_mesh,
    scratch_types=[
        pltpu.VMEM_SHARED((NI, D), jnp.float32),
        pltpu.VMEM((K, D), jnp.float32),
        pltpu.VMEM((NCHUNK, K), jnp.int32),
        pltpu.VMEM((R, D), jnp.float32),
    ],
)
def _deg_kernel(row3, out, acc, ones_v, idx_v, zbuf):
    c = lax.axis_index("c")
    s = lax.axis_index("s")
    _fill2d(zbuf, R, 0.0)
    _half_loop(c, s, lambda r0, rl: pltpu.sync_copy(zbuf, acc.at[pl.ds(rl, R)]))
    _fill2d(ones_v, K, 1.0)
    pltpu.sync_copy(row3.at[c, s], idx_v)
    plsc.subcore_barrier()

    def step(i, _):
        pltpu.sync_copy(ones_v, acc.at[idx_v.at[i]], add=True)
        return 0

    lax.fori_loop(0, NCHUNK, step, 0)
    plsc.subcore_barrier()

    def wout(r0, rl):
        pltpu.sync_copy(acc.at[pl.ds(rl, R)], zbuf)
        pltpu.sync_copy(zbuf, out.at[pl.ds(r0, R)])

    _half_loop(c, s, wout)


def _prep_body(degb, emb0, z0, invdb, sq6b):
    d = jnp.maximum(degb[...], 1.0)
    invdb[...] = 1.0 / d
    sq6b[...] = jnp.sqrt(d) * (1.0 / 6.0)
    z0[...] = emb0[...] * lax.rsqrt(d)


_prep_kernel = pl.pallas_call(
    _prep_body,
    out_shape=[
        jax.ShapeDtypeStruct((N, D), jnp.float32),
        jax.ShapeDtypeStruct((N, D), jnp.float32),
        jax.ShapeDtypeStruct((N, D), jnp.float32),
    ],
)


@functools.partial(
    pl.kernel,
    out_type=jax.ShapeDtypeStruct((N, D), jnp.float32),
    mesh=_mesh,
    scratch_types=[
        pltpu.VMEM_SHARED((NI, D), jnp.float32),
        pltpu.VMEM((NCHUNK, K), jnp.int32),
        pltpu.VMEM((NCHUNK, K), jnp.int32),
        pltpu.VMEM((K, D), jnp.float32),
        pltpu.VMEM((R, D), jnp.float32),
        pltpu.VMEM((R, D), jnp.float32),
        pltpu.SemaphoreType.DMA,
    ],
)
def _layer_kernel(z, col3, row3, invd16, znew, acc, col_v, row_v, rows, wbuf, ibuf, sem):
    c = lax.axis_index("c")
    s = lax.axis_index("s")

    _fill2d(wbuf, R, 0.0)
    _half_loop(c, s, lambda r0, rl: pltpu.sync_copy(wbuf, acc.at[pl.ds(rl, R)]))
    pltpu.sync_copy(col3.at[c, s], col_v)
    pltpu.sync_copy(row3.at[c, s], row_v)
    plsc.subcore_barrier()

    def step(i, _):
        pltpu.async_copy(z.at[col_v.at[i]], rows, sem).wait()
        pltpu.sync_copy(rows, acc.at[row_v.at[i]], add=True)
        return 0

    lax.fori_loop(0, NCHUNK, step, 0)
    plsc.subcore_barrier()

    def wout(r0, rl):
        pltpu.sync_copy(acc.at[pl.ds(rl, R)], wbuf)
        pltpu.sync_copy(invd16.at[pl.ds(r0, R)], ibuf)

        def srow(r, _):
            for gg in range(8):
                v = wbuf[r, pl.ds(gg * 16, 16)]
                wbuf[r, pl.ds(gg * 16, 16)] = v * ibuf[r, pl.ds(gg * 16, 16)]
            return 0

        lax.fori_loop(0, R, srow, 0)
        pltpu.sync_copy(wbuf, znew.at[pl.ds(r0, R)])

    _half_loop(c, s, wout)


@functools.partial(
    pl.kernel,
    out_type=jax.ShapeDtypeStruct((2048, D), jnp.float32),
    mesh=_mesh,
    scratch_types=[
        pltpu.VMEM((1, BIDX), jnp.int32),
        pltpu.VMEM((BIDX, D), jnp.float32),
        pltpu.VMEM((BIDX, D), jnp.float32),
        pltpu.VMEM((BIDX, D), jnp.float32),
        pltpu.SemaphoreType.DMA,
    ],
)
def _final_kernel(z0, z1, z2, z3, z4, z5, idx3, sq616, out, idx_v, acc_v, tmp_v, sq_v, sem):
    c = lax.axis_index("c")
    s = lax.axis_index("s")
    wid = c * NS + s
    pltpu.sync_copy(idx3.at[wid], idx_v)
    pltpu.async_copy(sq616.at[idx_v.at[0]], sq_v, sem).wait()
    pltpu.async_copy(z0.at[idx_v.at[0]], acc_v, sem).wait()
    for zl in (z1, z2, z3, z4, z5):
        pltpu.async_copy(zl.at[idx_v.at[0]], tmp_v, sem).wait()

        def addrow(r, _):
            for g in range(8):
                acc_v[r, pl.ds(g * 16, 16)] = (
                    acc_v[r, pl.ds(g * 16, 16)] + tmp_v[r, pl.ds(g * 16, 16)]
                )
            return 0

        lax.fori_loop(0, BIDX, addrow, 0)

    def srow(r, _):
        for g in range(8):
            acc_v[r, pl.ds(g * 16, 16)] = acc_v[r, pl.ds(g * 16, 16)] * sq_v[
                r, pl.ds(g * 16, 16)
            ]
        return 0

    lax.fori_loop(0, BIDX, srow, 0)
    pltpu.sync_copy(acc_v, out.at[pl.ds(wid * BIDX, BIDX)])


def kernel(users, items, user_table, item_table, adj_row, adj_col, adj_val):
    emb0 = jnp.concatenate([user_table, item_table], axis=0)
    row3 = adj_row.reshape(NC, NS, NCHUNK, K) - jnp.array(
        [0, NU], jnp.int32
    ).reshape(NC, 1, 1, 1)
    col3 = adj_col.reshape(NC, NS, NCHUNK, K)
    idx3 = jnp.concatenate([users, items + NU]).reshape(NC * NS, 1, BIDX)

    deg16 = _deg_kernel(row3)
    z0, invd, sq6 = _prep_kernel(deg16, emb0)

    zs = [z0]
    for _ in range(NLAYERS):
        zs.append(_layer_kernel(zs[-1], col3, row3, invd))

    outf = _final_kernel(zs[0], zs[1], zs[2], zs[3], zs[4], zs[5], idx3, sq6)
    return outf[:1024], outf[1024:]


# speedup vs baseline: 10.0049x; 1.5745x over previous
import functools

import jax
import jax.numpy as jnp
from jax import lax
from jax.experimental import pallas as pl
from jax.experimental.pallas import tpu as pltpu
from jax.experimental.pallas import tpu_sc as plsc

NU = 3000
NI = 7000
N = NU + NI
E = 320000
D = 128
NLAYERS = 5
NC = 2
NS = 16
K = 80
NCHUNK = E // (NC * NS * K)
R = 40
CH_U = NU // R
CH_I = NI // R
BIDX = 2048 // (NC * NS)

_mesh = plsc.VectorSubcoreMesh(
    core_axis_name="c", subcore_axis_name="s", num_cores=NC, num_subcores=NS
)


def _fill2d(ref, nrows, value):
    g = ref.shape[1] // 16

    def body(i, _):
        for j in range(g):
            ref[i, pl.ds(j * 16, 16)] = jnp.full((16,), value, jnp.float32)
        return 0

    lax.fori_loop(0, nrows, body, 0)


def _half_loop(c, s, body):
    base = c * NU
    nch = CH_U + c * (CH_I - CH_U)
    n = (nch - 1 - s) // NS + 1

    def wrap(j, _):
        rl = (s + j * NS) * R
        body(base + rl, rl)
        return 0

    lax.fori_loop(0, n, wrap, 0)


@functools.partial(
    pl.kernel,
    out_type=jax.ShapeDtypeStruct((N, D), jnp.float32),
    mesh=_mesh,
    scratch_types=[
        pltpu.VMEM_SHARED((NI, D), jnp.float32),
        pltpu.VMEM((K, D), jnp.float32),
        pltpu.VMEM((NCHUNK, K), jnp.int32),
        pltpu.VMEM((R, D), jnp.float32),
    ],
)
def _deg_kernel(row3, out, acc, ones_v, idx_v, zbuf):
    c = lax.axis_index("c")
    s = lax.axis_index("s")
    _fill2d(zbuf, R, 0.0)
    _half_loop(c, s, lambda r0, rl: pltpu.sync_copy(zbuf, acc.at[pl.ds(rl, R)]))
    _fill2d(ones_v, K, 1.0)
    pltpu.sync_copy(row3.at[c, s], idx_v)
    plsc.subcore_barrier()

    def step(i, _):
        pltpu.sync_copy(ones_v, acc.at[idx_v.at[i]], add=True)
        return 0

    lax.fori_loop(0, NCHUNK, step, 0)
    plsc.subcore_barrier()

    def wout(r0, rl):
        pltpu.sync_copy(acc.at[pl.ds(rl, R)], zbuf)
        pltpu.sync_copy(zbuf, out.at[pl.ds(r0, R)])

    _half_loop(c, s, wout)


def _prep_body(degb, emb0, z0, invdb, sq6b):
    d = jnp.maximum(degb[...], 1.0)
    invdb[...] = 1.0 / d
    sq6b[...] = jnp.sqrt(d) * (1.0 / 6.0)
    z0[...] = emb0[...] * lax.rsqrt(d)


_prep_kernel = pl.pallas_call(
    _prep_body,
    out_shape=[
        jax.ShapeDtypeStruct((N, D), jnp.float32),
        jax.ShapeDtypeStruct((N, D), jnp.float32),
        jax.ShapeDtypeStruct((N, D), jnp.float32),
    ],
)


@functools.partial(
    pl.kernel,
    out_type=jax.ShapeDtypeStruct((N, D), jnp.float32),
    mesh=_mesh,
    scratch_types=[
        pltpu.VMEM_SHARED((NI, D), jnp.float32),
        pltpu.VMEM((NCHUNK, K), jnp.int32),
        pltpu.VMEM((NCHUNK, K), jnp.int32),
        pltpu.VMEM((K, D), jnp.float32),
        pltpu.VMEM((K, D), jnp.float32),
        pltpu.VMEM((R, D), jnp.float32),
        pltpu.VMEM((R, D), jnp.float32),
        pltpu.SemaphoreType.DMA,
        pltpu.SemaphoreType.DMA,
        pltpu.SemaphoreType.DMA,
        pltpu.SemaphoreType.DMA,
    ],
)
def _layer_kernel(
    z, col3, row3, invd16, znew,
    acc, col_v, row_v, rows0, rows1, wbuf, ibuf, sg0, sg1, ss0, ss1,
):
    c = lax.axis_index("c")
    s = lax.axis_index("s")

    _fill2d(wbuf, R, 0.0)
    _half_loop(c, s, lambda r0, rl: pltpu.sync_copy(wbuf, acc.at[pl.ds(rl, R)]))
    pltpu.sync_copy(col3.at[c, s], col_v)
    pltpu.sync_copy(row3.at[c, s], row_v)
    plsc.subcore_barrier()

    def drain(buf, sem):
        pltpu.make_async_copy(z.at[pl.ds(0, K)], buf, sem).wait()

    pltpu.async_copy(z.at[col_v.at[0]], rows0, sg0)

    def pair(i, _):
        i0 = 2 * i
        pltpu.async_copy(z.at[col_v.at[i0 + 1]], rows1, sg1)
        drain(rows0, sg0)
        pltpu.async_copy(rows0, acc.at[row_v.at[i0]], ss0, add=True)
        drain(rows0, ss0)
        pltpu.async_copy(z.at[col_v.at[i0 + 2]], rows0, sg0)
        drain(rows1, sg1)
        pltpu.async_copy(rows1, acc.at[row_v.at[i0 + 1]], ss1, add=True)
        drain(rows1, ss1)
        return 0

    lax.fori_loop(0, (NCHUNK - 1) // 2, pair, 0)
    drain(rows0, sg0)
    pltpu.async_copy(rows0, acc.at[row_v.at[NCHUNK - 1]], ss0, add=True)
    drain(rows0, ss0)
    plsc.subcore_barrier()

    def wout(r0, rl):
        pltpu.sync_copy(acc.at[pl.ds(rl, R)], wbuf)
        pltpu.sync_copy(invd16.at[pl.ds(r0, R)], ibuf)

        def srow(r, _):
            for gg in range(8):
                v = wbuf[r, pl.ds(gg * 16, 16)]
                wbuf[r, pl.ds(gg * 16, 16)] = v * ibuf[r, pl.ds(gg * 16, 16)]
            return 0

        lax.fori_loop(0, R, srow, 0)
        pltpu.sync_copy(wbuf, znew.at[pl.ds(r0, R)])

    _half_loop(c, s, wout)


@functools.partial(
    pl.kernel,
    out_type=jax.ShapeDtypeStruct((2048, D), jnp.float32),
    mesh=_mesh,
    scratch_types=[
        pltpu.VMEM((1, BIDX), jnp.int32),
        pltpu.VMEM((BIDX, D), jnp.float32),
        pltpu.VMEM((BIDX, D), jnp.float32),
        pltpu.VMEM((BIDX, D), jnp.float32),
        pltpu.SemaphoreType.DMA,
    ],
)
def _final_kernel(z0, z1, z2, z3, z4, z5, idx3, sq616, out, idx_v, acc_v, tmp_v, sq_v, sem):
    c = lax.axis_index("c")
    s = lax.axis_index("s")
    wid = c * NS + s
    pltpu.sync_copy(idx3.at[wid], idx_v)
    pltpu.async_copy(sq616.at[idx_v.at[0]], sq_v, sem).wait()
    pltpu.async_copy(z0.at[idx_v.at[0]], acc_v, sem).wait()
    for zl in (z1, z2, z3, z4, z5):
        pltpu.async_copy(zl.at[idx_v.at[0]], tmp_v, sem).wait()

        def addrow(r, _):
            for g in range(8):
                acc_v[r, pl.ds(g * 16, 16)] = (
                    acc_v[r, pl.ds(g * 16, 16)] + tmp_v[r, pl.ds(g * 16, 16)]
                )
            return 0

        lax.fori_loop(0, BIDX, addrow, 0)

    def srow(r, _):
        for g in range(8):
            acc_v[r, pl.ds(g * 16, 16)] = acc_v[r, pl.ds(g * 16, 16)] * sq_v[
                r, pl.ds(g * 16, 16)
            ]
        return 0

    lax.fori_loop(0, BIDX, srow, 0)
    pltpu.sync_copy(acc_v, out.at[pl.ds(wid * BIDX, BIDX)])


def kernel(users, items, user_table, item_table, adj_row, adj_col, adj_val):
    emb0 = jnp.concatenate([user_table, item_table], axis=0)
    row3 = adj_row.reshape(NC, NS, NCHUNK, K) - jnp.array(
        [0, NU], jnp.int32
    ).reshape(NC, 1, 1, 1)
    col3 = adj_col.reshape(NC, NS, NCHUNK, K)
    idx3 = jnp.concatenate([users, items + NU]).reshape(NC * NS, 1, BIDX)

    deg16 = _deg_kernel(row3)
    z0, invd, sq6 = _prep_kernel(deg16, emb0)

    zs = [z0]
    for _ in range(NLAYERS):
        zs.append(_layer_kernel(zs[-1], col3, row3, invd))

    outf = _final_kernel(zs[0], zs[1], zs[2], zs[3], zs[4], zs[5], idx3, sq6)
    return outf[:1024], outf[1024:]


# speedup vs baseline: 10.9317x; 1.0926x over previous
import functools

import jax
import jax.numpy as jnp
from jax import lax
from jax.experimental import pallas as pl
from jax.experimental.pallas import tpu as pltpu
from jax.experimental.pallas import tpu_sc as plsc

NU = 3000
NI = 7000
N = NU + NI
E = 320000
D = 128
NLAYERS = 5
NC = 2
NS = 16
K = 125
NCHUNK = E // (NC * NS * K)
R = 40
CH_U = NU // R
CH_I = NI // R
BIDX = 2048 // (NC * NS)

_mesh = plsc.VectorSubcoreMesh(
    core_axis_name="c", subcore_axis_name="s", num_cores=NC, num_subcores=NS
)


def _fill2d(ref, nrows, value):
    g = ref.shape[1] // 16

    def body(i, _):
        for j in range(g):
            ref[i, pl.ds(j * 16, 16)] = jnp.full((16,), value, jnp.float32)
        return 0

    lax.fori_loop(0, nrows, body, 0)


def _half_loop(c, s, body):
    base = c * NU
    nch = CH_U + c * (CH_I - CH_U)
    n = (nch - 1 - s) // NS + 1

    def wrap(j, _):
        rl = (s + j * NS) * R
        body(base + rl, rl)
        return 0

    lax.fori_loop(0, n, wrap, 0)


@functools.partial(
    pl.kernel,
    out_type=jax.ShapeDtypeStruct((N, D), jnp.float32),
    mesh=_mesh,
    scratch_types=[
        pltpu.VMEM_SHARED((NI, D), jnp.float32),
        pltpu.VMEM((K, D), jnp.float32),
        pltpu.VMEM((NCHUNK, K), jnp.int32),
        pltpu.VMEM((R, D), jnp.float32),
    ],
)
def _deg_kernel(row3, out, acc, ones_v, idx_v, zbuf):
    c = lax.axis_index("c")
    s = lax.axis_index("s")
    _fill2d(zbuf, R, 0.0)
    _half_loop(c, s, lambda r0, rl: pltpu.sync_copy(zbuf, acc.at[pl.ds(rl, R)]))
    _fill2d(ones_v, K, 1.0)
    pltpu.sync_copy(row3.at[c, s], idx_v)
    plsc.subcore_barrier()

    def step(i, _):
        pltpu.sync_copy(ones_v, acc.at[idx_v.at[i]], add=True)
        return 0

    lax.fori_loop(0, NCHUNK, step, 0)
    plsc.subcore_barrier()

    def wout(r0, rl):
        pltpu.sync_copy(acc.at[pl.ds(rl, R)], zbuf)
        pltpu.sync_copy(zbuf, out.at[pl.ds(r0, R)])

    _half_loop(c, s, wout)


def _prep_body(degb, emb0, z0, invdb, sq6b):
    d = jnp.maximum(degb[...], 1.0)
    invdb[...] = 1.0 / d
    sq6b[...] = jnp.sqrt(d) * (1.0 / 6.0)
    z0[...] = emb0[...] * lax.rsqrt(d)


_prep_kernel = pl.pallas_call(
    _prep_body,
    out_shape=[
        jax.ShapeDtypeStruct((N, D), jnp.float32),
        jax.ShapeDtypeStruct((N, D), jnp.float32),
        jax.ShapeDtypeStruct((N, D), jnp.float32),
    ],
)


@functools.partial(
    pl.kernel,
    out_type=jax.ShapeDtypeStruct((N, D), jnp.float32),
    mesh=_mesh,
    scratch_types=[
        pltpu.VMEM_SHARED((NI, D), jnp.float32),
        pltpu.VMEM((NCHUNK, K), jnp.int32),
        pltpu.VMEM((NCHUNK, K), jnp.int32),
        pltpu.VMEM((K, D), jnp.float32),
        pltpu.VMEM((K, D), jnp.float32),
        pltpu.VMEM((R, D), jnp.float32),
        pltpu.VMEM((R, D), jnp.float32),
        pltpu.SemaphoreType.DMA,
        pltpu.SemaphoreType.DMA,
        pltpu.SemaphoreType.DMA,
        pltpu.SemaphoreType.DMA,
    ],
)
def _layer_kernel(
    z, col3, row3, invd16, znew,
    acc, col_v, row_v, rows0, rows1, wbuf, ibuf, sg0, sg1, ss0, ss1,
):
    c = lax.axis_index("c")
    s = lax.axis_index("s")

    _fill2d(wbuf, R, 0.0)
    _half_loop(c, s, lambda r0, rl: pltpu.sync_copy(wbuf, acc.at[pl.ds(rl, R)]))
    pltpu.sync_copy(col3.at[c, s], col_v)
    pltpu.sync_copy(row3.at[c, s], row_v)
    plsc.subcore_barrier()

    def drain(buf, sem):
        pltpu.make_async_copy(z.at[col_v.at[0]], buf, sem).wait()

    pltpu.async_copy(z.at[col_v.at[0]], rows0, sg0)

    def pair(i, _):
        i0 = 2 * i
        pltpu.async_copy(z.at[col_v.at[i0 + 1]], rows1, sg1)
        drain(rows0, sg0)
        pltpu.async_copy(rows0, acc.at[row_v.at[i0]], ss0, add=True)
        drain(rows0, ss0)
        pltpu.async_copy(z.at[col_v.at[i0 + 2]], rows0, sg0)
        drain(rows1, sg1)
        pltpu.async_copy(rows1, acc.at[row_v.at[i0 + 1]], ss1, add=True)
        drain(rows1, ss1)
        return 0

    lax.fori_loop(0, (NCHUNK - 2) // 2, pair, 0)
    pltpu.async_copy(z.at[col_v.at[NCHUNK - 1]], rows1, sg1)
    drain(rows0, sg0)
    pltpu.async_copy(rows0, acc.at[row_v.at[NCHUNK - 2]], ss0, add=True)
    drain(rows1, sg1)
    pltpu.async_copy(rows1, acc.at[row_v.at[NCHUNK - 1]], ss1, add=True)
    drain(rows0, ss0)
    drain(rows1, ss1)
    plsc.subcore_barrier()

    def wout(r0, rl):
        pltpu.sync_copy(acc.at[pl.ds(rl, R)], wbuf)
        pltpu.sync_copy(invd16.at[pl.ds(r0, R)], ibuf)

        def srow(r, _):
            for gg in range(8):
                v = wbuf[r, pl.ds(gg * 16, 16)]
                wbuf[r, pl.ds(gg * 16, 16)] = v * ibuf[r, pl.ds(gg * 16, 16)]
            return 0

        lax.fori_loop(0, R, srow, 0)
        pltpu.sync_copy(wbuf, znew.at[pl.ds(r0, R)])

    _half_loop(c, s, wout)


@functools.partial(
    pl.kernel,
    out_type=jax.ShapeDtypeStruct((2048, D), jnp.float32),
    mesh=_mesh,
    scratch_types=[
        pltpu.VMEM((1, BIDX), jnp.int32),
        pltpu.VMEM((BIDX, D), jnp.float32),
        pltpu.VMEM((BIDX, D), jnp.float32),
        pltpu.VMEM((BIDX, D), jnp.float32),
        pltpu.SemaphoreType.DMA,
    ],
)
def _final_kernel(z0, z1, z2, z3, z4, z5, idx3, sq616, out, idx_v, acc_v, tmp_v, sq_v, sem):
    c = lax.axis_index("c")
    s = lax.axis_index("s")
    wid = c * NS + s
    pltpu.sync_copy(idx3.at[wid], idx_v)
    pltpu.async_copy(sq616.at[idx_v.at[0]], sq_v, sem).wait()
    pltpu.async_copy(z0.at[idx_v.at[0]], acc_v, sem).wait()
    for zl in (z1, z2, z3, z4, z5):
        pltpu.async_copy(zl.at[idx_v.at[0]], tmp_v, sem).wait()

        def addrow(r, _):
            for g in range(8):
                acc_v[r, pl.ds(g * 16, 16)] = (
                    acc_v[r, pl.ds(g * 16, 16)] + tmp_v[r, pl.ds(g * 16, 16)]
                )
            return 0

        lax.fori_loop(0, BIDX, addrow, 0)

    def srow(r, _):
        for g in range(8):
            acc_v[r, pl.ds(g * 16, 16)] = acc_v[r, pl.ds(g * 16, 16)] * sq_v[
                r, pl.ds(g * 16, 16)
            ]
        return 0

    lax.fori_loop(0, BIDX, srow, 0)
    pltpu.sync_copy(acc_v, out.at[pl.ds(wid * BIDX, BIDX)])


def kernel(users, items, user_table, item_table, adj_row, adj_col, adj_val):
    emb0 = jnp.concatenate([user_table, item_table], axis=0)
    row3 = adj_row.reshape(NC, NS, NCHUNK, K) - jnp.array(
        [0, NU], jnp.int32
    ).reshape(NC, 1, 1, 1)
    col3 = adj_col.reshape(NC, NS, NCHUNK, K)
    idx3 = jnp.concatenate([users, items + NU]).reshape(NC * NS, 1, BIDX)

    deg16 = _deg_kernel(row3)
    z0, invd, sq6 = _prep_kernel(deg16, emb0)

    zs = [z0]
    for _ in range(NLAYERS):
        zs.append(_layer_kernel(zs[-1], col3, row3, invd))

    outf = _final_kernel(zs[0], zs[1], zs[2], zs[3], zs[4], zs[5], idx3, sq6)
    return outf[:1024], outf[1024:]


# speedup vs baseline: 11.2434x; 1.0285x over previous
import functools

import jax
import jax.numpy as jnp
from jax import lax
from jax.experimental import pallas as pl
from jax.experimental.pallas import tpu as pltpu
from jax.experimental.pallas import tpu_sc as plsc

NU = 3000
NI = 7000
N = NU + NI
E = 320000
D = 128
NLAYERS = 5
NC = 2
NS = 16
K = 125
NCHUNK = E // (NC * NS * K)
R = 40
CH_U = NU // R
CH_I = NI // R
BIDX = 2048 // (NC * NS)

_mesh = plsc.VectorSubcoreMesh(
    core_axis_name="c", subcore_axis_name="s", num_cores=NC, num_subcores=NS
)


def _fill2d(ref, nrows, value):
    g = ref.shape[1] // 16

    def body(i, _):
        for j in range(g):
            ref[i, pl.ds(j * 16, 16)] = jnp.full((16,), value, jnp.float32)
        return 0

    lax.fori_loop(0, nrows, body, 0)


def _half_loop(c, s, body):
    base = c * NU
    nch = CH_U + c * (CH_I - CH_U)
    n = (nch - 1 - s) // NS + 1

    def wrap(j, _):
        rl = (s + j * NS) * R
        body(base + rl, rl)
        return 0

    lax.fori_loop(0, n, wrap, 0)


@functools.partial(
    pl.kernel,
    out_type=jax.ShapeDtypeStruct((N, D), jnp.float32),
    mesh=_mesh,
    scratch_types=[
        pltpu.VMEM_SHARED((NI, D), jnp.float32),
        pltpu.VMEM((K, D), jnp.float32),
        pltpu.VMEM((NCHUNK, K), jnp.int32),
        pltpu.VMEM((R, D), jnp.float32),
    ],
)
def _deg_kernel(row3, out, acc, ones_v, idx_v, zbuf):
    c = lax.axis_index("c")
    s = lax.axis_index("s")
    _fill2d(zbuf, R, 0.0)
    _half_loop(c, s, lambda r0, rl: pltpu.sync_copy(zbuf, acc.at[pl.ds(rl, R)]))
    _fill2d(ones_v, K, 1.0)
    pltpu.sync_copy(row3.at[c, s], idx_v)
    plsc.subcore_barrier()

    def step(i, _):
        pltpu.sync_copy(ones_v, acc.at[idx_v.at[i]], add=True)
        return 0

    lax.fori_loop(0, NCHUNK, step, 0)
    plsc.subcore_barrier()

    def wout(r0, rl):
        pltpu.sync_copy(acc.at[pl.ds(rl, R)], zbuf)
        pltpu.sync_copy(zbuf, out.at[pl.ds(r0, R)])

    _half_loop(c, s, wout)


def _prep_body(degb, emb0, z0, invdb, sq6b):
    d = jnp.maximum(degb[...], 1.0)
    invdb[...] = 1.0 / d
    sq6b[...] = jnp.sqrt(d) * (1.0 / 6.0)
    z0[...] = emb0[...] * lax.rsqrt(d)


_prep_kernel = pl.pallas_call(
    _prep_body,
    out_shape=[
        jax.ShapeDtypeStruct((N, D), jnp.float32),
        jax.ShapeDtypeStruct((N, D), jnp.float32),
        jax.ShapeDtypeStruct((N, D), jnp.float32),
    ],
)


@functools.partial(
    pl.kernel,
    out_type=jax.ShapeDtypeStruct((N, D), jnp.float32),
    mesh=_mesh,
    scratch_types=[
        pltpu.VMEM_SHARED((NI, D), jnp.float32),
        pltpu.VMEM((NCHUNK, K), jnp.int32),
        pltpu.VMEM((NCHUNK, K), jnp.int32),
        pltpu.VMEM((K, D), jnp.float32),
        pltpu.VMEM((K, D), jnp.float32),
        pltpu.VMEM((K, D), jnp.float32),
        pltpu.SemaphoreType.DMA,
        pltpu.SemaphoreType.DMA,
        pltpu.SemaphoreType.DMA,
        pltpu.SemaphoreType.DMA,
        pltpu.SemaphoreType.DMA,
        pltpu.SemaphoreType.DMA,
    ],
)
def _layer_kernel(
    z, col3, row3, invd16, znew,
    acc, col_v, row_v, rows0, rows1, rows2, sg0, sg1, sg2, ss0, ss1, ss2,
):
    c = lax.axis_index("c")
    s = lax.axis_index("s")

    _fill2d(rows2, R, 0.0)
    _half_loop(
        c, s,
        lambda r0, rl: pltpu.sync_copy(rows2.at[pl.ds(0, R)], acc.at[pl.ds(rl, R)]),
    )
    pltpu.sync_copy(col3.at[c, s], col_v)
    pltpu.sync_copy(row3.at[c, s], row_v)
    plsc.subcore_barrier()

    def drain(buf, sem):
        pltpu.make_async_copy(z.at[col_v.at[0]], buf, sem).wait()

    def gather(i, buf, sem):
        pltpu.async_copy(z.at[col_v.at[i]], buf, sem)

    def scatter(i, buf, sem):
        pltpu.async_copy(buf, acc.at[row_v.at[i]], sem, add=True)

    gather(0, rows0, sg0)
    gather(1, rows1, sg1)

    def ring(i, _):
        i0 = 3 * i
        drain(rows0, sg0)
        scatter(i0, rows0, ss0)
        gather(i0 + 2, rows2, sg2)
        drain(rows1, sg1)
        scatter(i0 + 1, rows1, ss1)
        drain(rows0, ss0)
        gather(i0 + 3, rows0, sg0)
        drain(rows2, sg2)
        scatter(i0 + 2, rows2, ss2)
        drain(rows1, ss1)
        gather(i0 + 4, rows1, sg1)
        drain(rows2, ss2)
        return 0

    lax.fori_loop(0, (NCHUNK - 2) // 3, ring, 0)
    drain(rows0, sg0)
    scatter(NCHUNK - 2, rows0, ss0)
    drain(rows1, sg1)
    scatter(NCHUNK - 1, rows1, ss1)
    drain(rows0, ss0)
    drain(rows1, ss1)
    plsc.subcore_barrier()

    def wout(r0, rl):
        pltpu.sync_copy(acc.at[pl.ds(rl, R)], rows2.at[pl.ds(0, R)])
        pltpu.sync_copy(invd16.at[pl.ds(r0, R)], rows1.at[pl.ds(0, R)])

        def srow(r, _):
            for gg in range(8):
                v = rows2[r, pl.ds(gg * 16, 16)]
                rows2[r, pl.ds(gg * 16, 16)] = v * rows1[r, pl.ds(gg * 16, 16)]
            return 0

        lax.fori_loop(0, R, srow, 0)
        pltpu.sync_copy(rows2.at[pl.ds(0, R)], znew.at[pl.ds(r0, R)])

    _half_loop(c, s, wout)


@functools.partial(
    pl.kernel,
    out_type=jax.ShapeDtypeStruct((2048, D), jnp.float32),
    mesh=_mesh,
    scratch_types=[
        pltpu.VMEM((1, BIDX), jnp.int32),
        pltpu.VMEM((BIDX, D), jnp.float32),
        pltpu.VMEM((BIDX, D), jnp.float32),
        pltpu.VMEM((BIDX, D), jnp.float32),
        pltpu.SemaphoreType.DMA,
    ],
)
def _final_kernel(z0, z1, z2, z3, z4, z5, idx3, sq616, out, idx_v, acc_v, tmp_v, sq_v, sem):
    c = lax.axis_index("c")
    s = lax.axis_index("s")
    wid = c * NS + s
    pltpu.sync_copy(idx3.at[wid], idx_v)
    pltpu.async_copy(sq616.at[idx_v.at[0]], sq_v, sem).wait()
    pltpu.async_copy(z0.at[idx_v.at[0]], acc_v, sem).wait()
    for zl in (z1, z2, z3, z4, z5):
        pltpu.async_copy(zl.at[idx_v.at[0]], tmp_v, sem).wait()

        def addrow(r, _):
            for g in range(8):
                acc_v[r, pl.ds(g * 16, 16)] = (
                    acc_v[r, pl.ds(g * 16, 16)] + tmp_v[r, pl.ds(g * 16, 16)]
                )
            return 0

        lax.fori_loop(0, BIDX, addrow, 0)

    def srow(r, _):
        for g in range(8):
            acc_v[r, pl.ds(g * 16, 16)] = acc_v[r, pl.ds(g * 16, 16)] * sq_v[
                r, pl.ds(g * 16, 16)
            ]
        return 0

    lax.fori_loop(0, BIDX, srow, 0)
    pltpu.sync_copy(acc_v, out.at[pl.ds(wid * BIDX, BIDX)])


def kernel(users, items, user_table, item_table, adj_row, adj_col, adj_val):
    emb0 = jnp.concatenate([user_table, item_table], axis=0)
    row3 = adj_row.reshape(NC, NS, NCHUNK, K) - jnp.array(
        [0, NU], jnp.int32
    ).reshape(NC, 1, 1, 1)
    col3 = adj_col.reshape(NC, NS, NCHUNK, K)
    idx3 = jnp.concatenate([users, items + NU]).reshape(NC * NS, 1, BIDX)

    deg16 = _deg_kernel(row3)
    z0, invd, sq6 = _prep_kernel(deg16, emb0)

    zs = [z0]
    for _ in range(NLAYERS):
        zs.append(_layer_kernel(zs[-1], col3, row3, invd))

    outf = _final_kernel(zs[0], zs[1], zs[2], zs[3], zs[4], zs[5], idx3, sq6)
    return outf[:1024], outf[1024:]


# speedup vs baseline: 12.1202x; 1.0780x over previous
import functools

import jax
import jax.numpy as jnp
from jax import lax
from jax.experimental import pallas as pl
from jax.experimental.pallas import tpu as pltpu
from jax.experimental.pallas import tpu_sc as plsc

NU = 3000
NI = 7000
N = NU + NI
E = 320000
D = 128
NLAYERS = 5
NC = 2
NS = 16
K = 125
NCHUNK = E // (NC * NS * K)
R = 40
CH_U = NU // R
CH_I = NI // R
BIDX = 2048 // (NC * NS)
EPT = E // (NC * NS)

_mesh = plsc.VectorSubcoreMesh(
    core_axis_name="c", subcore_axis_name="s", num_cores=NC, num_subcores=NS
)


def _fill2d(ref, nrows, value):
    g = ref.shape[1] // 16

    def body(i, _):
        for j in range(g):
            ref[i, pl.ds(j * 16, 16)] = jnp.full((16,), value, jnp.float32)
        return 0

    lax.fori_loop(0, nrows, body, 0)


def _half_loop(c, s, body):
    base = c * NU
    nch = CH_U + c * (CH_I - CH_U)
    n = (nch - 1 - s) // NS + 1

    def wrap(j, _):
        rl = (s + j * NS) * R
        body(base + rl, rl)
        return 0

    lax.fori_loop(0, n, wrap, 0)


@functools.partial(
    pl.kernel,
    out_type=jax.ShapeDtypeStruct((NC, 56, D), jnp.float32),
    mesh=_mesh,
    compiler_params=pltpu.CompilerParams(needs_layout_passes=False),
    scratch_types=[
        pltpu.VMEM_SHARED((56, D), jnp.float32),
        pltpu.VMEM((EPT,), jnp.int32),
        pltpu.VMEM((56 * D,), jnp.float32),
        pltpu.VMEM((56, D), jnp.float32),
        pltpu.VMEM((64,), jnp.int32),
    ],
)
def _deg_kernel(rowflat, out, acc, idx_v, part, part2, idxr):
    c = lax.axis_index("c")
    s = lax.axis_index("s")
    wid = c * NS + s

    def zrow(i, _):
        part[pl.ds(i * 16, 16)] = jnp.zeros((16,), jnp.float32)
        return 0

    lax.fori_loop(0, 56 * D // 16, zrow, 0)

    @pl.when(s == 0)
    def _():
        def z2(i, _):
            for j in range(8):
                part2[i, pl.ds(j * 16, 16)] = jnp.zeros((16,), jnp.float32)
            return 0

        lax.fori_loop(0, 56, z2, 0)
        pltpu.sync_copy(part2, acc)


    for i in range(4):
        idxr[pl.ds(i * 16, 16)] = lax.iota(jnp.int32, 16) + (i * 16)
    pltpu.sync_copy(rowflat.at[pl.ds(wid * EPT, EPT)], idx_v)
    plsc.subcore_barrier()

    ones = jnp.ones((16,), jnp.float32)

    def step(i, _):
        iv = idx_v[pl.ds(i * 16, 16)]
        plsc.addupdate_scatter(part, [iv], ones)
        return 0

    lax.fori_loop(0, EPT // 16, step, 0)
    def mv(i, _):
        for j in range(8):
            part2[i, pl.ds(j * 16, 16)] = part[pl.ds(i * D + j * 16, 16)]
        return 0

    lax.fori_loop(0, 56, mv, 0)
    pltpu.sync_copy(part2, acc.at[idxr.at[pl.ds(0, 56)]], add=True)
    plsc.subcore_barrier()

    @pl.when(s == 0)
    def _():
        pltpu.sync_copy(acc, out.at[c])


def _prep_body(dcol, emb0, z0, invdb, sq6b):
    d = jnp.maximum(dcol[...], 1.0)
    invdb[...] = jnp.broadcast_to(1.0 / d, (N, D))
    sq6b[...] = jnp.broadcast_to(jnp.sqrt(d) * (1.0 / 6.0), (N, D))
    z0[...] = emb0[...] * lax.rsqrt(d)


_prep_kernel = pl.pallas_call(
    _prep_body,
    out_shape=[
        jax.ShapeDtypeStruct((N, D), jnp.float32),
        jax.ShapeDtypeStruct((N, D), jnp.float32),
        jax.ShapeDtypeStruct((N, D), jnp.float32),
    ],
)


@functools.partial(
    pl.kernel,
    out_type=jax.ShapeDtypeStruct((N, D), jnp.float32),
    mesh=_mesh,
    scratch_types=[
        pltpu.VMEM_SHARED((NI, D), jnp.float32),
        pltpu.VMEM((NCHUNK, K), jnp.int32),
        pltpu.VMEM((NCHUNK, K), jnp.int32),
        pltpu.VMEM((K, D), jnp.float32),
        pltpu.VMEM((K, D), jnp.float32),
        pltpu.VMEM((K, D), jnp.float32),
        pltpu.SemaphoreType.DMA,
        pltpu.SemaphoreType.DMA,
        pltpu.SemaphoreType.DMA,
        pltpu.SemaphoreType.DMA,
        pltpu.SemaphoreType.DMA,
        pltpu.SemaphoreType.DMA,
    ],
)
def _layer_kernel(
    z, col3, row3, invd16, znew,
    acc, col_v, row_v, rows0, rows1, rows2, sg0, sg1, sg2, ss0, ss1, ss2,
):
    c = lax.axis_index("c")
    s = lax.axis_index("s")

    _fill2d(rows2, R, 0.0)
    _half_loop(
        c, s,
        lambda r0, rl: pltpu.sync_copy(rows2.at[pl.ds(0, R)], acc.at[pl.ds(rl, R)]),
    )
    pltpu.sync_copy(col3.at[c, s], col_v)
    pltpu.sync_copy(row3.at[c, s], row_v)
    plsc.subcore_barrier()

    def drain(buf, sem):
        pltpu.make_async_copy(z.at[col_v.at[0]], buf, sem).wait()

    def gather(i, buf, sem):
        pltpu.async_copy(z.at[col_v.at[i]], buf, sem)

    def scatter(i, buf, sem):
        pltpu.async_copy(buf, acc.at[row_v.at[i]], sem, add=True)

    gather(0, rows0, sg0)
    gather(1, rows1, sg1)

    def ring(i, _):
        i0 = 3 * i
        drain(rows0, sg0)
        scatter(i0, rows0, ss0)
        gather(i0 + 2, rows2, sg2)
        drain(rows1, sg1)
        scatter(i0 + 1, rows1, ss1)
        drain(rows0, ss0)
        gather(i0 + 3, rows0, sg0)
        drain(rows2, sg2)
        scatter(i0 + 2, rows2, ss2)
        drain(rows1, ss1)
        gather(i0 + 4, rows1, sg1)
        drain(rows2, ss2)
        return 0

    lax.fori_loop(0, (NCHUNK - 2) // 3, ring, 0)
    drain(rows0, sg0)
    scatter(NCHUNK - 2, rows0, ss0)
    drain(rows1, sg1)
    scatter(NCHUNK - 1, rows1, ss1)
    drain(rows0, ss0)
    drain(rows1, ss1)
    plsc.subcore_barrier()

    def wout(r0, rl):
        pltpu.sync_copy(acc.at[pl.ds(rl, R)], rows2.at[pl.ds(0, R)])
        pltpu.sync_copy(invd16.at[pl.ds(r0, R)], rows1.at[pl.ds(0, R)])

        def srow(r, _):
            for gg in range(8):
                v = rows2[r, pl.ds(gg * 16, 16)]
                rows2[r, pl.ds(gg * 16, 16)] = v * rows1[r, pl.ds(gg * 16, 16)]
            return 0

        lax.fori_loop(0, R, srow, 0)
        pltpu.sync_copy(rows2.at[pl.ds(0, R)], znew.at[pl.ds(r0, R)])

    _half_loop(c, s, wout)


@functools.partial(
    pl.kernel,
    out_type=jax.ShapeDtypeStruct((2048, D), jnp.float32),
    mesh=_mesh,
    scratch_types=[
        pltpu.VMEM((1, BIDX), jnp.int32),
        pltpu.VMEM((BIDX, D), jnp.float32),
        pltpu.VMEM((BIDX, D), jnp.float32),
        pltpu.VMEM((BIDX, D), jnp.float32),
        pltpu.SemaphoreType.DMA,
    ],
)
def _final_kernel(z0, z1, z2, z3, z4, z5, idx3, sq616, out, idx_v, acc_v, tmp_v, sq_v, sem):
    c = lax.axis_index("c")
    s = lax.axis_index("s")
    wid = c * NS + s
    pltpu.sync_copy(idx3.at[wid], idx_v)
    pltpu.async_copy(sq616.at[idx_v.at[0]], sq_v, sem).wait()
    pltpu.async_copy(z0.at[idx_v.at[0]], acc_v, sem).wait()
    for zl in (z1, z2, z3, z4, z5):
        pltpu.async_copy(zl.at[idx_v.at[0]], tmp_v, sem).wait()

        def addrow(r, _):
            for g in range(8):
                acc_v[r, pl.ds(g * 16, 16)] = (
                    acc_v[r, pl.ds(g * 16, 16)] + tmp_v[r, pl.ds(g * 16, 16)]
                )
            return 0

        lax.fori_loop(0, BIDX, addrow, 0)

    def srow(r, _):
        for g in range(8):
            acc_v[r, pl.ds(g * 16, 16)] = acc_v[r, pl.ds(g * 16, 16)] * sq_v[
                r, pl.ds(g * 16, 16)
            ]
        return 0

    lax.fori_loop(0, BIDX, srow, 0)
    pltpu.sync_copy(acc_v, out.at[pl.ds(wid * BIDX, BIDX)])


def kernel(users, items, user_table, item_table, adj_row, adj_col, adj_val):
    emb0 = jnp.concatenate([user_table, item_table], axis=0)
    row3 = adj_row.reshape(NC, NS, NCHUNK, K) - jnp.array(
        [0, NU], jnp.int32
    ).reshape(NC, 1, 1, 1)
    col3 = adj_col.reshape(NC, NS, NCHUNK, K)
    idx3 = jnp.concatenate([users, items + NU]).reshape(NC * NS, 1, BIDX)

    degh = _deg_kernel(row3.reshape(E))
    dcol = jnp.concatenate(
        [degh[0].reshape(56 * D)[:NU], degh[1].reshape(56 * D)[:NI]]
    ).reshape(N, 1)
    z0, invd, sq6 = _prep_kernel(dcol, emb0)

    zs = [z0]
    for _ in range(NLAYERS):
        zs.append(_layer_kernel(zs[-1], col3, row3, invd))

    outf = _final_kernel(zs[0], zs[1], zs[2], zs[3], zs[4], zs[5], idx3, sq6)
    return outf[:1024], outf[1024:]


# speedup vs baseline: 12.8866x; 1.0632x over previous
import functools

import jax
import jax.numpy as jnp
from jax import lax
from jax.experimental import pallas as pl
from jax.experimental.pallas import tpu as pltpu
from jax.experimental.pallas import tpu_sc as plsc

NU = 3000
NI = 7000
N = NU + NI
E = 320000
D = 128
NLAYERS = 5
NC = 2
NS = 16
K = 125
NCHUNK = E // (NC * NS * K)
R = 40
CH_U = NU // R
CH_I = NI // R
BIDX = 2048 // (NC * NS)
EPT = E // (NC * NS)

_mesh = plsc.VectorSubcoreMesh(
    core_axis_name="c", subcore_axis_name="s", num_cores=NC, num_subcores=NS
)


def _fill2d(ref, nrows, value):
    g = ref.shape[1] // 16

    def body(i, _):
        for j in range(g):
            ref[i, pl.ds(j * 16, 16)] = jnp.full((16,), value, jnp.float32)
        return 0

    lax.fori_loop(0, nrows, body, 0)


def _half_loop(c, s, body):
    base = c * NU
    nch = CH_U + c * (CH_I - CH_U)
    n = (nch - 1 - s) // NS + 1

    def wrap(j, _):
        rl = (s + j * NS) * R
        body(base + rl, rl)
        return 0

    lax.fori_loop(0, n, wrap, 0)


@functools.partial(
    pl.kernel,
    out_type=jax.ShapeDtypeStruct((NC, 56, D), jnp.float32),
    mesh=_mesh,
    compiler_params=pltpu.CompilerParams(needs_layout_passes=False),
    scratch_types=[
        pltpu.VMEM_SHARED((56, D), jnp.float32),
        pltpu.VMEM((EPT,), jnp.int32),
        pltpu.VMEM((56 * D,), jnp.float32),
        pltpu.VMEM((56, D), jnp.float32),
        pltpu.VMEM((64,), jnp.int32),
    ],
)
def _deg_kernel(rowflat, out, acc, idx_v, part, part2, idxr):
    c = lax.axis_index("c")
    s = lax.axis_index("s")
    wid = c * NS + s

    def zrow(i, _):
        part[pl.ds(i * 16, 16)] = jnp.zeros((16,), jnp.float32)
        return 0

    lax.fori_loop(0, 56 * D // 16, zrow, 0)

    @pl.when(s == 0)
    def _():
        def z2(i, _):
            for j in range(8):
                part2[i, pl.ds(j * 16, 16)] = jnp.zeros((16,), jnp.float32)
            return 0

        lax.fori_loop(0, 56, z2, 0)
        pltpu.sync_copy(part2, acc)


    for i in range(4):
        idxr[pl.ds(i * 16, 16)] = lax.iota(jnp.int32, 16) + (i * 16)
    pltpu.sync_copy(rowflat.at[pl.ds(wid * EPT, EPT)], idx_v)
    plsc.subcore_barrier()

    ones = jnp.ones((16,), jnp.float32)

    def step(i, _):
        iv = idx_v[pl.ds(i * 16, 16)]
        plsc.addupdate_scatter(part, [iv], ones)
        return 0

    lax.fori_loop(0, EPT // 16, step, 0)
    def mv(i, _):
        for j in range(8):
            part2[i, pl.ds(j * 16, 16)] = part[pl.ds(i * D + j * 16, 16)]
        return 0

    lax.fori_loop(0, 56, mv, 0)
    pltpu.sync_copy(part2, acc.at[idxr.at[pl.ds(0, 56)]], add=True)
    plsc.subcore_barrier()

    @pl.when(s == 0)
    def _():
        pltpu.sync_copy(acc, out.at[c])


def _prep_body(dcol, emb0, z0, invdb, sq6b):
    d = jnp.maximum(dcol[...], 1.0)
    invdb[...] = jnp.broadcast_to(1.0 / d, (N, D))
    sq6b[...] = jnp.broadcast_to(jnp.sqrt(d) * (1.0 / 6.0), (N, D))
    z0[...] = emb0[...] * lax.rsqrt(d)


_prep_kernel = pl.pallas_call(
    _prep_body,
    out_shape=[
        jax.ShapeDtypeStruct((N, D), jnp.float32),
        jax.ShapeDtypeStruct((N, D), jnp.float32),
        jax.ShapeDtypeStruct((N, D), jnp.float32),
    ],
)


@functools.partial(
    pl.kernel,
    out_type=jax.ShapeDtypeStruct((N, D), jnp.float32),
    mesh=_mesh,
    scratch_types=[
        pltpu.VMEM_SHARED((NI, D), jnp.float32),
        pltpu.VMEM((NCHUNK, K), jnp.int32),
        pltpu.VMEM((NCHUNK, K), jnp.int32),
        pltpu.VMEM((K, D), jnp.float32),
        pltpu.VMEM((K, D), jnp.float32),
        pltpu.VMEM((K, D), jnp.float32),
        pltpu.SemaphoreType.DMA,
        pltpu.SemaphoreType.DMA,
        pltpu.SemaphoreType.DMA,
        pltpu.SemaphoreType.DMA,
        pltpu.SemaphoreType.DMA,
        pltpu.SemaphoreType.DMA,
    ],
)
def _layer_kernel(
    z, col3, row3, invd16, znew,
    acc, col_v, row_v, rows0, rows1, rows2, sg0, sg1, sg2, ss0, ss1, ss2,
):
    c = lax.axis_index("c")
    s = lax.axis_index("s")

    pltpu.async_copy(col3.at[c, s], col_v, sg0)
    pltpu.async_copy(row3.at[c, s], row_v, sg1)
    _fill2d(rows2, R, 0.0)
    _half_loop(
        c, s,
        lambda r0, rl: pltpu.async_copy(
            rows2.at[pl.ds(0, R)], acc.at[pl.ds(rl, R)], ss0
        ),
    )
    _half_loop(
        c, s,
        lambda r0, rl: pltpu.make_async_copy(
            invd16.at[pl.ds(0, R)], rows2.at[pl.ds(0, R)], ss0
        ).wait(),
    )
    pltpu.make_async_copy(col3.at[c, s], col_v, sg0).wait()
    pltpu.make_async_copy(row3.at[c, s], row_v, sg1).wait()
    plsc.subcore_barrier()

    def drain(buf, sem):
        pltpu.make_async_copy(z.at[col_v.at[0]], buf, sem).wait()

    def gather(i, buf, sem):
        pltpu.async_copy(z.at[col_v.at[i]], buf, sem)

    def scatter(i, buf, sem):
        pltpu.async_copy(buf, acc.at[row_v.at[i]], sem, add=True)

    gather(0, rows0, sg0)
    gather(1, rows1, sg1)

    def ring(i, _):
        i0 = 3 * i
        drain(rows0, sg0)
        scatter(i0, rows0, ss0)
        gather(i0 + 2, rows2, sg2)
        drain(rows1, sg1)
        scatter(i0 + 1, rows1, ss1)
        drain(rows0, ss0)
        gather(i0 + 3, rows0, sg0)
        drain(rows2, sg2)
        scatter(i0 + 2, rows2, ss2)
        drain(rows1, ss1)
        gather(i0 + 4, rows1, sg1)
        drain(rows2, ss2)
        return 0

    lax.fori_loop(0, (NCHUNK - 2) // 3, ring, 0)
    drain(rows0, sg0)
    scatter(NCHUNK - 2, rows0, ss0)
    drain(rows1, sg1)
    scatter(NCHUNK - 1, rows1, ss1)
    drain(rows0, ss0)
    drain(rows1, ss1)
    plsc.subcore_barrier()

    base = c * NU
    nch = CH_U + c * (CH_I - CH_U)
    nw = (nch - 1 - s) // NS + 1

    def wdrain(buf, sem):
        pltpu.make_async_copy(
            invd16.at[pl.ds(0, R)], buf.at[pl.ds(0, R)], sem
        ).wait()

    def wchunk(j, _):
        rl = (s + j * NS) * R
        r0 = base + rl
        pltpu.async_copy(acc.at[pl.ds(rl, R)], rows2.at[pl.ds(0, R)], sg0)
        pltpu.async_copy(invd16.at[pl.ds(r0, R)], rows1.at[pl.ds(0, R)], sg1)

        @pl.when(j > 0)
        def _():
            wdrain(rows0, ss1)

        wdrain(rows2, sg0)
        wdrain(rows1, sg1)

        def srow(r, _):
            for gg in range(8):
                rows0[r, pl.ds(gg * 16, 16)] = (
                    rows2[r, pl.ds(gg * 16, 16)] * rows1[r, pl.ds(gg * 16, 16)]
                )
            return 0

        lax.fori_loop(0, R, srow, 0)
        pltpu.async_copy(rows0.at[pl.ds(0, R)], znew.at[pl.ds(r0, R)], ss1)
        return 0

    lax.fori_loop(0, nw, wchunk, 0)
    wdrain(rows0, ss1)


@functools.partial(
    pl.kernel,
    out_type=jax.ShapeDtypeStruct((2048, D), jnp.float32),
    mesh=_mesh,
    scratch_types=[
        pltpu.VMEM((1, BIDX), jnp.int32),
        pltpu.VMEM((BIDX, D), jnp.float32),
        pltpu.VMEM((BIDX, D), jnp.float32),
        pltpu.VMEM((BIDX, D), jnp.float32),
        pltpu.VMEM((BIDX, D), jnp.float32),
        pltpu.VMEM((BIDX, D), jnp.float32),
        pltpu.VMEM((BIDX, D), jnp.float32),
        pltpu.VMEM((BIDX, D), jnp.float32),
        pltpu.SemaphoreType.DMA,
    ],
)
def _final_kernel(
    z0, z1, z2, z3, z4, z5, idx3, sq616, out,
    idx_v, b0, b1, b2, b3, b4, b5, sq_v, sem,
):
    c = lax.axis_index("c")
    s = lax.axis_index("s")
    wid = c * NS + s
    pltpu.sync_copy(idx3.at[wid], idx_v)
    for zl, b in ((z0, b0), (z1, b1), (z2, b2), (z3, b3), (z4, b4), (z5, b5)):
        pltpu.async_copy(zl.at[idx_v.at[0]], b, sem)
    pltpu.async_copy(sq616.at[idx_v.at[0]], sq_v, sem)
    for _ in range(7):
        pltpu.make_async_copy(z0.at[idx_v.at[0]], b0, sem).wait()

    def srow(r, _):
        for g in range(8):
            sl = pl.ds(g * 16, 16)
            b0[r, sl] = (
                ((b0[r, sl] + b1[r, sl]) + (b2[r, sl] + b3[r, sl]))
                + (b4[r, sl] + b5[r, sl])
            ) * sq_v[r, sl]
        return 0

    lax.fori_loop(0, BIDX, srow, 0)
    pltpu.sync_copy(b0, out.at[pl.ds(wid * BIDX, BIDX)])


def kernel(users, items, user_table, item_table, adj_row, adj_col, adj_val):
    emb0 = jnp.concatenate([user_table, item_table], axis=0)
    row3 = adj_row.reshape(NC, NS, NCHUNK, K) - jnp.array(
        [0, NU], jnp.int32
    ).reshape(NC, 1, 1, 1)
    col3 = adj_col.reshape(NC, NS, NCHUNK, K)
    idx3 = jnp.concatenate([users, items + NU]).reshape(NC * NS, 1, BIDX)

    degh = _deg_kernel(row3.reshape(E))
    dcol = jnp.concatenate(
        [degh[0].reshape(56 * D)[:NU], degh[1].reshape(56 * D)[:NI]]
    ).reshape(N, 1)
    z0, invd, sq6 = _prep_kernel(dcol, emb0)

    zs = [z0]
    for _ in range(NLAYERS):
        zs.append(_layer_kernel(zs[-1], col3, row3, invd))

    outf = _final_kernel(zs[0], zs[1], zs[2], zs[3], zs[4], zs[5], idx3, sq6)
    return outf[:1024], outf[1024:]


# speedup vs baseline: 13.2218x; 1.0260x over previous
import functools

import jax
import jax.numpy as jnp
from jax import lax
from jax.experimental import pallas as pl
from jax.experimental.pallas import tpu as pltpu
from jax.experimental.pallas import tpu_sc as plsc

NU = 3000
NI = 7000
N = NU + NI
E = 320000
D = 128
NLAYERS = 5
NC = 2
NS = 16
K = 125
NCHUNK = E // (NC * NS * K)
R = 40
CH_U = NU // R
CH_I = NI // R
BIDX = 2048 // (NC * NS)
EPT = E // (NC * NS)

_mesh = plsc.VectorSubcoreMesh(
    core_axis_name="c", subcore_axis_name="s", num_cores=NC, num_subcores=NS
)


def _fill2d(ref, nrows, value):
    g = ref.shape[1] // 16

    def body(i, _):
        for j in range(g):
            ref[i, pl.ds(j * 16, 16)] = jnp.full((16,), value, jnp.float32)
        return 0

    lax.fori_loop(0, nrows, body, 0)


def _half_loop(c, s, body):
    base = c * NU
    nch = CH_U + c * (CH_I - CH_U)
    n = (nch - 1 - s) // NS + 1

    def wrap(j, _):
        rl = (s + j * NS) * R
        body(base + rl, rl)
        return 0

    lax.fori_loop(0, n, wrap, 0)


@functools.partial(
    pl.kernel,
    out_type=jax.ShapeDtypeStruct((NC, 56, D), jnp.float32),
    mesh=_mesh,
    compiler_params=pltpu.CompilerParams(needs_layout_passes=False),
    scratch_types=[
        pltpu.VMEM_SHARED((56, D), jnp.float32),
        pltpu.VMEM((EPT,), jnp.int32),
        pltpu.VMEM((56 * D,), jnp.float32),
        pltpu.VMEM((56, D), jnp.float32),
        pltpu.VMEM((64,), jnp.int32),
    ],
)
def _deg_kernel(rowflat, out, acc, idx_v, part, part2, idxr):
    c = lax.axis_index("c")
    s = lax.axis_index("s")
    wid = c * NS + s

    def zrow(i, _):
        part[pl.ds(i * 16, 16)] = jnp.zeros((16,), jnp.float32)
        return 0

    lax.fori_loop(0, 56 * D // 16, zrow, 0)

    @pl.when(s == 0)
    def _():
        def z2(i, _):
            for j in range(8):
                part2[i, pl.ds(j * 16, 16)] = jnp.zeros((16,), jnp.float32)
            return 0

        lax.fori_loop(0, 56, z2, 0)
        pltpu.sync_copy(part2, acc)


    for i in range(4):
        idxr[pl.ds(i * 16, 16)] = lax.iota(jnp.int32, 16) + (i * 16)
    pltpu.sync_copy(rowflat.at[pl.ds(wid * EPT, EPT)], idx_v)
    plsc.subcore_barrier()

    ones = jnp.ones((16,), jnp.float32)

    def step(i, _):
        iv = idx_v[pl.ds(i * 16, 16)]
        plsc.addupdate_scatter(part, [iv], ones)
        return 0

    lax.fori_loop(0, EPT // 16, step, 0)
    def mv(i, _):
        for j in range(8):
            part2[i, pl.ds(j * 16, 16)] = part[pl.ds(i * D + j * 16, 16)]
        return 0

    lax.fori_loop(0, 56, mv, 0)
    pltpu.sync_copy(part2, acc.at[idxr.at[pl.ds(0, 56)]], add=True)
    plsc.subcore_barrier()

    @pl.when(s == 0)
    def _():
        pltpu.sync_copy(acc, out.at[c])


def _prep_body(dcol, emb0, z0, invdb, sq6b):
    d = jnp.maximum(dcol[...], 1.0)
    invdb[...] = jnp.broadcast_to(1.0 / d, (N, D))
    sq6b[...] = jnp.broadcast_to(jnp.sqrt(d) * (1.0 / 6.0), (N, D))
    z0[...] = emb0[...] * lax.rsqrt(d)


_prep_kernel = pl.pallas_call(
    _prep_body,
    out_shape=[
        jax.ShapeDtypeStruct((N, D), jnp.float32),
        jax.ShapeDtypeStruct((N, D), jnp.float32),
        jax.ShapeDtypeStruct((N, D), jnp.float32),
    ],
)


@functools.partial(
    pl.kernel,
    out_type=[jax.ShapeDtypeStruct((2048, D), jnp.float32)]
    + [jax.ShapeDtypeStruct((N, D), jnp.float32)] * NLAYERS,
    mesh=_mesh,
    scratch_types=[
        pltpu.VMEM_SHARED((NI, D), jnp.float32),
        pltpu.VMEM((NCHUNK, K), jnp.int32),
        pltpu.VMEM((NCHUNK, K), jnp.int32),
        pltpu.VMEM((K, D), jnp.float32),
        pltpu.VMEM((K, D), jnp.float32),
        pltpu.VMEM((K, D), jnp.float32),
        pltpu.VMEM((1, BIDX), jnp.int32),
        pltpu.SemaphoreType.DMA,
        pltpu.SemaphoreType.DMA,
        pltpu.SemaphoreType.DMA,
        pltpu.SemaphoreType.DMA,
        pltpu.SemaphoreType.DMA,
        pltpu.SemaphoreType.DMA,
        pltpu.SemaphoreType.REGULAR,
    ],
)
def _layers_kernel(
    z0, col3, row3, invd16, idx3, sq616,
    out, z1, z2, z3, z4, z5,
    acc, col_v, row_v, rows0, rows1, rows2, idx_v,
    sg0, sg1, sg2, ss0, ss1, ss2, bsem,
):
    c = lax.axis_index("c")
    s = lax.axis_index("s")
    wid = c * NS + s

    pltpu.async_copy(col3.at[c, s], col_v, sg0)
    pltpu.async_copy(row3.at[c, s], row_v, sg1)
    pltpu.async_copy(idx3.at[wid], idx_v, sg2)
    pltpu.make_async_copy(col3.at[c, s], col_v, sg0).wait()
    pltpu.make_async_copy(row3.at[c, s], row_v, sg1).wait()
    pltpu.make_async_copy(idx3.at[wid], idx_v, sg2).wait()

    def drain(buf, sem):
        pltpu.make_async_copy(z0.at[col_v.at[0]], buf, sem).wait()

    def zero_acc():
        _fill2d(rows2, R, 0.0)
        _half_loop(
            c, s,
            lambda r0, rl: pltpu.async_copy(
                rows2.at[pl.ds(0, R)], acc.at[pl.ds(rl, R)], ss0
            ),
        )
        _half_loop(
            c, s,
            lambda r0, rl: pltpu.make_async_copy(
                invd16.at[pl.ds(0, R)], rows2.at[pl.ds(0, R)], ss0
            ).wait(),
        )

    def edge_phase(z):
        def gather(i, buf, sem):
            pltpu.async_copy(z.at[col_v.at[i]], buf, sem)

        def scatter(i, buf, sem):
            pltpu.async_copy(buf, acc.at[row_v.at[i]], sem, add=True)

        gather(0, rows0, sg0)
        gather(1, rows1, sg1)

        def ring(i, _):
            i0 = 3 * i
            drain(rows0, sg0)
            scatter(i0, rows0, ss0)
            gather(i0 + 2, rows2, sg2)
            drain(rows1, sg1)
            scatter(i0 + 1, rows1, ss1)
            drain(rows0, ss0)
            gather(i0 + 3, rows0, sg0)
            drain(rows2, sg2)
            scatter(i0 + 2, rows2, ss2)
            drain(rows1, ss1)
            gather(i0 + 4, rows1, sg1)
            drain(rows2, ss2)
            return 0

        lax.fori_loop(0, (NCHUNK - 2) // 3, ring, 0)
        drain(rows0, sg0)
        scatter(NCHUNK - 2, rows0, ss0)
        drain(rows1, sg1)
        scatter(NCHUNK - 1, rows1, ss1)
        drain(rows0, ss0)
        drain(rows1, ss1)

    def writeout(znew):
        base = c * NU
        nch = CH_U + c * (CH_I - CH_U)
        nw = (nch - 1 - s) // NS + 1

        def wdrain(buf, sem):
            pltpu.make_async_copy(
                invd16.at[pl.ds(0, R)], buf.at[pl.ds(0, R)], sem
            ).wait()

        def wchunk(j, _):
            rl = (s + j * NS) * R
            r0 = base + rl
            pltpu.async_copy(acc.at[pl.ds(rl, R)], rows2.at[pl.ds(0, R)], sg0)
            pltpu.async_copy(invd16.at[pl.ds(r0, R)], rows1.at[pl.ds(0, R)], sg1)

            @pl.when(j > 0)
            def _():
                wdrain(rows0, ss1)

            wdrain(rows2, sg0)
            wdrain(rows1, sg1)

            def srow(r, _):
                for gg in range(8):
                    rows0[r, pl.ds(gg * 16, 16)] = (
                        rows2[r, pl.ds(gg * 16, 16)] * rows1[r, pl.ds(gg * 16, 16)]
                    )
                return 0

            lax.fori_loop(0, R, srow, 0)
            pltpu.async_copy(rows0.at[pl.ds(0, R)], znew.at[pl.ds(r0, R)], ss1)
            return 0

        lax.fori_loop(0, nw, wchunk, 0)
        wdrain(rows0, ss1)

    zsrcs = (z0, z1, z2, z3, z4)
    zdsts = (z1, z2, z3, z4, z5)
    for l in range(NLAYERS):
        zero_acc()
        plsc.subcore_barrier()
        pltpu.core_barrier(bsem, core_axis_name="c")
        edge_phase(zsrcs[l])
        plsc.subcore_barrier()
        writeout(zdsts[l])

    plsc.subcore_barrier()
    pltpu.core_barrier(bsem, core_axis_name="c")

    def g64(srcref, buf, sem):
        pltpu.async_copy(srcref.at[idx_v.at[0]], buf.at[pl.ds(0, BIDX)], sem)

    def d64(buf, sem):
        pltpu.make_async_copy(
            z0.at[idx_v.at[0]], buf.at[pl.ds(0, BIDX)], sem
        ).wait()

    def combine(src2, mul):
        def srow(r, _):
            for g in range(8):
                sl = pl.ds(g * 16, 16)
                if mul:
                    rows0[r, sl] = rows0[r, sl] * src2[r, sl]
                else:
                    rows0[r, sl] = rows0[r, sl] + src2[r, sl]
            return 0

        lax.fori_loop(0, BIDX, srow, 0)

    g64(z0, rows0, sg0)
    g64(z1, rows1, sg1)
    g64(z2, rows2, sg2)
    d64(rows0, sg0)
    d64(rows1, sg1)
    combine(rows1, False)
    g64(z3, rows1, sg1)
    d64(rows2, sg2)
    combine(rows2, False)
    g64(z4, rows2, sg2)
    d64(rows1, sg1)
    combine(rows1, False)
    g64(z5, rows1, sg1)
    d64(rows2, sg2)
    combine(rows2, False)
    g64(sq616, rows2, sg2)
    d64(rows1, sg1)
    combine(rows1, False)
    d64(rows2, sg2)
    combine(rows2, True)
    pltpu.sync_copy(rows0.at[pl.ds(0, BIDX)], out.at[pl.ds(wid * BIDX, BIDX)])


def kernel(users, items, user_table, item_table, adj_row, adj_col, adj_val):
    emb0 = jnp.concatenate([user_table, item_table], axis=0)
    row3 = adj_row.reshape(NC, NS, NCHUNK, K) - jnp.array(
        [0, NU], jnp.int32
    ).reshape(NC, 1, 1, 1)
    col3 = adj_col.reshape(NC, NS, NCHUNK, K)
    idx3 = jnp.concatenate([users, items + NU]).reshape(NC * NS, 1, BIDX)

    degh = _deg_kernel(row3.reshape(E))
    dcol = jnp.concatenate(
        [degh[0].reshape(56 * D)[:NU], degh[1].reshape(56 * D)[:NI]]
    ).reshape(N, 1)
    z0, invd, sq6 = _prep_kernel(dcol, emb0)

    outf = _layers_kernel(z0, col3, row3, invd, idx3, sq6)[0]
    return outf[:1024], outf[1024:]
